# Initial kernel scaffold; baseline (speedup 1.0000x reference)
import jax
import jax.numpy as jnp
from jax.experimental import pallas as pl


def kernel(velocity):
    raise NotImplementedError("write your pallas kernel here")


# speedup vs baseline: 4.8338x; 4.8338x over previous
import functools

import jax
import jax.numpy as jnp
from jax import lax
from jax.experimental import pallas as pl
from jax.experimental.pallas import tpu as pltpu
from jax.experimental.pallas import tpu_sc as plsc

D = 128
N = D * D * D
TIME_STEP = 5

NC, NS, L = 2, 16, 16
NW = NC * NS
PER_W = N // NW
C = 1024
SLICES = C // 128
GROUPS = C // L
N_CHUNKS = PER_W // C

_F32 = jnp.float32
_I32 = jnp.int32


def _splat_i(v):
    return jnp.full((L,), v, _I32)


def _splat_f(v):
    return jnp.full((L,), v, _F32)


def _make_step(scale: float):
    mesh = plsc.VectorSubcoreMesh(
        core_axis_name="c", subcore_axis_name="s", num_cores=NC, num_subcores=NS
    )

    cpos = scale * (D - 1) / 2.0

    @functools.partial(
        pl.kernel,
        out_type=tuple(jax.ShapeDtypeStruct((N,), _F32) for _ in range(3)),
        mesh=mesh,
        scratch_types=[
            tuple(pltpu.VMEM((C,), _F32) for _ in range(3)),
            tuple(pltpu.VMEM((C,), _F32) for _ in range(3)),
            tuple(pltpu.VMEM((8 * C,), _F32) for _ in range(3)),
            pltpu.VMEM((8, SLICES, 128), _I32),
            pltpu.VMEM((8 * C,), _F32),
            pltpu.SemaphoreType.DMA,
        ],
    )
    def step(t0, t1, t2, o0, o1, o2, inb, outb, gb, idxbuf, wbuf, sem):
        tins = (t0, t1, t2)
        touts = (o0, o1, o2)
        wid = lax.axis_index("s") * NC + lax.axis_index("c")
        lane = lax.iota(_I32, L)

        def chunk_body(j, _):
            rowbase = wid * PER_W + j * C

            for c in range(3):
                pltpu.sync_copy(tins[c].at[pl.ds(rowbase, C)], inb[c])

            def wgt_body(g, _):
                voxbase = g * L
                vi = _splat_i(0) + voxbase + lane
                p = vi + rowbase

                i0 = lax.shift_right_logical(p, 14)
                i1 = lax.bitwise_and(lax.shift_right_logical(p, 7), _splat_i(127))
                i2 = lax.bitwise_and(p, _splat_i(127))

                f0 = inb[0][pl.ds(voxbase, L)]
                f1 = inb[1][pl.ds(voxbase, L)]
                f2 = inb[2][pl.ds(voxbase, L)]

                zero = _splat_f(0.0)
                hi = _splat_f(float(D - 1))

                def axis_terms(i_int, f):
                    pos = i_int.astype(_F32) + f * cpos
                    pos = jnp.minimum(jnp.maximum(pos, zero), hi)
                    b = jnp.minimum(pos.astype(_I32), _splat_i(D - 2))
                    t = pos - b.astype(_F32)
                    return b, t

                b0, t0v = axis_terms(i0, f0)
                b1, t1v = axis_terms(i1, f1)
                b2, t2v = axis_terms(i2, f2)

                one = _splat_f(1.0)
                u0, u1, u2 = one - t0v, one - t1v, one - t2v

                base_idx = (
                    lax.shift_left(b0, _splat_i(14))
                    + lax.shift_left(b1, _splat_i(7))
                    + b2
                )

                a00 = u0 * u1
                a01 = u0 * t1v
                a10 = t0v * u1
                a11 = t0v * t1v

                srow = lax.shift_right_logical(voxbase, 7)
                scol = lax.rem(voxbase, jnp.int32(128))

                offs = (0, 1, 128, 129, 16384, 16385, 16512, 16513)
                wts = (a00 * u2, a00 * t2v, a01 * u2, a01 * t2v,
                       a10 * u2, a10 * t2v, a11 * u2, a11 * t2v)
                for k in range(8):
                    idxbuf[k, srow, pl.ds(scol, L)] = base_idx + _splat_i(offs[k])
                    wbuf[pl.ds(k * C + voxbase, L)] = wts[k]
                return 0

            lax.fori_loop(0, GROUPS, wgt_body, 0)

            copies = []
            for k in range(8):
                for s in range(SLICES):
                    for c in range(3):
                        copies.append(pltpu.async_copy(
                            tins[c].at[idxbuf.at[k, s]],
                            gb[c].at[pl.ds(k * C + s * 128, 128)],
                            sem,
                        ))
            for cp in copies:
                cp.wait()

            def mac_body(g, _):
                vb = g * L
                ws = [wbuf[pl.ds(k * C + vb, L)] for k in range(8)]
                for c in range(3):
                    acc = inb[c][pl.ds(vb, L)]
                    for k in range(8):
                        acc = acc + gb[c][pl.ds(k * C + vb, L)] * ws[k]
                    if scale != 1.0:
                        acc = acc * _splat_f(scale)
                    outb[c][pl.ds(vb, L)] = acc
                return 0

            lax.fori_loop(0, GROUPS, mac_body, 0)

            for c in range(3):
                pltpu.sync_copy(outb[c], touts[c].at[pl.ds(rowbase, C)])
            return 0

        lax.fori_loop(0, N_CHUNKS, chunk_body, 0)

    return step


def kernel(velocity):
    t0, t1, t2 = (velocity[:, c] for c in range(3))
    step1 = _make_step(1.0 / (2.0 ** TIME_STEP))
    stepn = _make_step(1.0)
    t0, t1, t2 = step1(t0, t1, t2)
    for _ in range(TIME_STEP - 1):
        t0, t1, t2 = stepn(t0, t1, t2)
    return jnp.stack([t0, t1, t2], axis=1)
